# baseline (speedup 1.0000x reference)
import functools

import jax
import jax.numpy as jnp
from jax import lax
from jax.experimental import pallas as pl
from jax.experimental.pallas import tpu as pltpu
from jax.experimental.pallas import tpu_sc as plsc

_PATCH = 200
_STRIDE = 150
_MAXP = 1000
_T = 150050
_NC, _NS = 2, 16
_NW = _NC * _NS
_K = 40
_NCHUNK = _MAXP // _K
_INLEN = _K * _STRIDE + 64
_OUTLEN = _K * _PATCH
_VOFF = tuple(16 * j for j in range(12)) + (184,)


def _sc_body(x_hbm, out_hbm, in_v, out_v, sem):
    w = lax.axis_index("s") * _NC + lax.axis_index("c")
    in_base = w * _T
    out_base = w * _MAXP * _PATCH
    shift = (w * _T) % 8

    def chunk(ci, _):
        p0 = ci * _K
        off = pl.multiple_of(in_base + _STRIDE * p0 - shift, 8)
        pltpu.sync_copy(x_hbm.at[pl.ds(off, _INLEN)], in_v)
        for k in range(_K):
            src = _STRIDE * k + shift
            dst = _PATCH * k
            for off in _VOFF:
                out_v[pl.ds(dst + off, 16)] = in_v[pl.ds(src + off, 16)]
        off = pl.multiple_of(out_base + _PATCH * p0, 8)
        pltpu.sync_copy(out_v, out_hbm.at[pl.ds(off, _OUTLEN)])
        return 0

    lax.fori_loop(0, _NCHUNK, chunk, 0)


@jax.jit
def _extract_patches(x1):
    mesh = plsc.VectorSubcoreMesh(core_axis_name="c", subcore_axis_name="s")
    return pl.kernel(
        _sc_body,
        out_type=jax.ShapeDtypeStruct((_NW * _MAXP * _PATCH,), jnp.float32),
        mesh=mesh,
        scratch_types=[
            pltpu.VMEM((_INLEN,), jnp.float32),
            pltpu.VMEM((_OUTLEN,), jnp.float32),
            pltpu.SemaphoreType.DMA,
        ],
        compiler_params=pltpu.CompilerParams(use_tc_tiling_on_sc=False),
    )(x1)


def kernel(x):
    B, C, T = x.shape
    assert (B * C, T) == (_NW, _T)
    x1 = x.reshape(-1)
    out = _extract_patches(x1)
    patches = out.reshape(B, C, _MAXP, _PATCH)
    masks = jnp.ones((B, C, _MAXP), jnp.float32)
    return (patches, masks)


# speedup vs baseline: 1.0527x; 1.0527x over previous
import functools

import jax
import jax.numpy as jnp
from jax import lax
from jax.experimental import pallas as pl
from jax.experimental.pallas import tpu as pltpu
from jax.experimental.pallas import tpu_sc as plsc

_PATCH = 200
_STRIDE = 150
_MAXP = 1000
_T = 150050
_NC, _NS = 2, 16
_NW = _NC * _NS
_K = 40
_NCHUNK = _MAXP // _K
_INLEN = _K * _STRIDE + 56
_OUTLEN = _K * _PATCH
_VOFF = tuple(16 * j for j in range(12)) + (184,)


def _sc_body(x_hbm, out_hbm, in_v, out_v, sem):
    w = lax.axis_index("s") * _NC + lax.axis_index("c")
    out_base = w * _MAXP * _PATCH

    def chunk(ci, _):
        p0 = ci * _K
        pltpu.sync_copy(x_hbm.at[w, pl.ds(_STRIDE * p0, _INLEN)], in_v)
        for k in range(_K):
            src = _STRIDE * k
            dst = _PATCH * k
            for off in _VOFF:
                out_v[pl.ds(dst + off, 16)] = in_v[pl.ds(src + off, 16)]
        o = pl.multiple_of(out_base + _PATCH * p0, 8)
        pltpu.sync_copy(out_v, out_hbm.at[pl.ds(o, _OUTLEN)])
        return 0

    lax.fori_loop(0, _NCHUNK, chunk, 0)


@jax.jit
def _extract_patches(x2):
    mesh = plsc.VectorSubcoreMesh(core_axis_name="c", subcore_axis_name="s")
    return pl.kernel(
        _sc_body,
        out_type=jax.ShapeDtypeStruct((_NW * _MAXP * _PATCH,), jnp.float32),
        mesh=mesh,
        scratch_types=[
            pltpu.VMEM((_INLEN,), jnp.float32),
            pltpu.VMEM((_OUTLEN,), jnp.float32),
            pltpu.SemaphoreType.DMA,
        ],
        compiler_params=pltpu.CompilerParams(use_tc_tiling_on_sc=False),
    )(x2)


def kernel(x):
    B, C, T = x.shape
    assert (B * C, T) == (_NW, _T)
    x2 = x.reshape(_NW, _T)
    out = _extract_patches(x2)
    patches = out.reshape(B, C, _MAXP, _PATCH)
    masks = jnp.ones((B, C, _MAXP), jnp.float32)
    return (patches, masks)
